# Initial kernel scaffold; baseline (speedup 1.0000x reference)
import jax
import jax.numpy as jnp
from jax.experimental import pallas as pl


def kernel(position_ids, hidden_states, ln1_w, Wqkv, b_qkv, sinks, Wo, b_o, ln2_w, Wr, br, w1, b1, w2, b2):
    raise NotImplementedError("write your pallas kernel here")


# speedup vs baseline: 1.9239x; 1.9239x over previous
import functools
import math

import jax
import jax.numpy as jnp
from jax import lax
from jax.experimental import pallas as pl
from jax.experimental.pallas import tpu as pltpu

T = 2048
HIDDEN = 1024
N_HEADS = 16
N_KV = 4
HEAD_DIM = 64
Q_SIZE = N_HEADS * HEAD_DIM
KV_SIZE = N_KV * HEAD_DIM
E = 16
TOP_K = 2
INTER = 1024
EPS = 1e-06
ROPE_THETA = 150000.0
FACTOR = 32.0
ORIG_MAX = 4096
BETA_FAST = 32.0
BETA_SLOW = 1.0
WINDOW = 128

TB = 256
NTB = T // TB
MB = 128
N_PAD = TOP_K * T + E * MB
NMB = N_PAD // MB
HALF = HEAD_DIM // 2


def _yarn_inv_freq():
    half = HEAD_DIM // 2
    pos_freqs = ROPE_THETA ** (jnp.arange(0, half, dtype=jnp.float32) * 2.0 / HEAD_DIM)
    inv_freq = 1.0 / pos_freqs
    inv_freq_inter = inv_freq / FACTOR

    def corr_dim(r):
        return HEAD_DIM * math.log(ORIG_MAX / (r * 2.0 * math.pi)) / (2.0 * math.log(ROPE_THETA))

    low = max(math.floor(corr_dim(BETA_FAST)), 0)
    high = min(math.ceil(corr_dim(BETA_SLOW)), half - 1)
    ramp = jnp.clip((jnp.arange(half, dtype=jnp.float32) - low) / max(high - low, 1), 0.0, 1.0)
    mask = 1.0 - ramp
    return inv_freq * mask + inv_freq_inter * (1.0 - mask)


def _qkv_body(hs, ln1, wqkv, bqkv, cos, sin, q_out, k_out, v_out):
    x = hs[...]
    var = jnp.mean(x * x, axis=-1, keepdims=True)
    h = x * lax.rsqrt(var + EPS) * ln1[...]
    qkv = jnp.dot(h, wqkv[...], preferred_element_type=jnp.float32) + bqkv[...]
    c = cos[...]
    s = sin[...]
    for hd in range(N_HEADS):
        base = hd * HEAD_DIM
        x1 = qkv[:, base:base + HALF]
        x2 = qkv[:, base + HALF:base + HEAD_DIM]
        q_out[:, base:base + HALF] = x1 * c - x2 * s
        q_out[:, base + HALF:base + HEAD_DIM] = x2 * c + x1 * s
    for hd in range(N_KV):
        base = Q_SIZE + hd * HEAD_DIM
        ob = hd * HEAD_DIM
        x1 = qkv[:, base:base + HALF]
        x2 = qkv[:, base + HALF:base + HEAD_DIM]
        k_out[:, ob:ob + HALF] = x1 * c - x2 * s
        k_out[:, ob + HALF:ob + HEAD_DIM] = x2 * c + x1 * s
    v_out[...] = qkv[:, Q_SIZE + KV_SIZE:]


def _qkv_call(hidden_states, ln1_w, Wqkv, b_qkv, cos, sin):
    return pl.pallas_call(
        _qkv_body,
        grid=(NTB,),
        in_specs=[
            pl.BlockSpec((TB, HIDDEN), lambda i: (i, 0)),
            pl.BlockSpec((1, HIDDEN), lambda i: (0, 0)),
            pl.BlockSpec((HIDDEN, Q_SIZE + 2 * KV_SIZE), lambda i: (0, 0)),
            pl.BlockSpec((1, Q_SIZE + 2 * KV_SIZE), lambda i: (0, 0)),
            pl.BlockSpec((TB, HALF), lambda i: (i, 0)),
            pl.BlockSpec((TB, HALF), lambda i: (i, 0)),
        ],
        out_specs=[
            pl.BlockSpec((TB, Q_SIZE), lambda i: (i, 0)),
            pl.BlockSpec((TB, KV_SIZE), lambda i: (i, 0)),
            pl.BlockSpec((TB, KV_SIZE), lambda i: (i, 0)),
        ],
        out_shape=[
            jax.ShapeDtypeStruct((T, Q_SIZE), jnp.float32),
            jax.ShapeDtypeStruct((T, KV_SIZE), jnp.float32),
            jax.ShapeDtypeStruct((T, KV_SIZE), jnp.float32),
        ],
    )(hidden_states, ln1_w, Wqkv, b_qkv, cos, sin)


def _attn_body(q, kp, kc, kn, vp, vc, vn, sinks, out):
    qb = pl.program_id(0)
    scale = HEAD_DIM ** -0.5
    i_abs = qb * TB + lax.broadcasted_iota(jnp.int32, (TB, 3 * 128), 0)
    j_abs = (2 * qb - 1) * 128 + lax.broadcasted_iota(jnp.int32, (TB, 3 * 128), 1)
    mask = (j_abs >= 0) & (j_abs <= i_abs) & (i_abs - j_abs < WINDOW)
    for hd in range(N_HEADS):
        hk = hd // (N_HEADS // N_KV)
        qh = q[:, hd * HEAD_DIM:(hd + 1) * HEAD_DIM]
        kh = jnp.concatenate(
            [kp[:, hk * HEAD_DIM:(hk + 1) * HEAD_DIM],
             kc[:, hk * HEAD_DIM:(hk + 1) * HEAD_DIM],
             kn[:, hk * HEAD_DIM:(hk + 1) * HEAD_DIM]], axis=0)
        vh = jnp.concatenate(
            [vp[:, hk * HEAD_DIM:(hk + 1) * HEAD_DIM],
             vc[:, hk * HEAD_DIM:(hk + 1) * HEAD_DIM],
             vn[:, hk * HEAD_DIM:(hk + 1) * HEAD_DIM]], axis=0)
        scores = lax.dot_general(qh, kh, (((1,), (1,)), ((), ())),
                                 preferred_element_type=jnp.float32) * scale
        scores = jnp.where(mask, scores, -1e30)
        sink = sinks[0:1, hd:hd + 1]
        m = jnp.maximum(jnp.max(scores, axis=-1, keepdims=True), sink)
        e = jnp.exp(scores - m)
        denom = jnp.sum(e, axis=-1, keepdims=True) + jnp.exp(sink - m)
        probs = e / denom
        out[:, hd * HEAD_DIM:(hd + 1) * HEAD_DIM] = jnp.dot(
            probs, vh, preferred_element_type=jnp.float32)


def _attn_call(q, k, v, sinks):
    kv_spec_p = pl.BlockSpec((128, KV_SIZE), lambda i: (jnp.maximum(2 * i - 1, 0), 0))
    kv_spec_c = pl.BlockSpec((128, KV_SIZE), lambda i: (2 * i, 0))
    kv_spec_n = pl.BlockSpec((128, KV_SIZE), lambda i: (2 * i + 1, 0))
    return pl.pallas_call(
        _attn_body,
        grid=(NTB,),
        in_specs=[
            pl.BlockSpec((TB, Q_SIZE), lambda i: (i, 0)),
            kv_spec_p, kv_spec_c, kv_spec_n,
            kv_spec_p, kv_spec_c, kv_spec_n,
            pl.BlockSpec((1, N_HEADS), lambda i: (0, 0)),
        ],
        out_specs=pl.BlockSpec((TB, Q_SIZE), lambda i: (i, 0)),
        out_shape=jax.ShapeDtypeStruct((T, Q_SIZE), jnp.float32),
    )(q, k, k, k, v, v, v, sinks)


def _post_body(attn, wo, bo, hs, ln2, wr, br,
               res2_out, h2_out, e0_out, e1_out, r0_out, r1_out,
               w0_out, w1_out, cnt_out, cnt_s):
    i = pl.program_id(0)

    @pl.when(i == 0)
    def _():
        cnt_s[...] = jnp.zeros_like(cnt_s)

    o = jnp.dot(attn[...], wo[...], preferred_element_type=jnp.float32) + bo[...]
    res2 = o + hs[...]
    res2_out[...] = res2
    var = jnp.mean(res2 * res2, axis=-1, keepdims=True)
    h2 = res2 * lax.rsqrt(var + EPS) * ln2[...]
    h2_out[...] = h2
    g = jnp.dot(h2, wr[...], preferred_element_type=jnp.float32) + br[...]

    iota = lax.broadcasted_iota(jnp.int32, (TB, E), 1)
    v1 = jnp.max(g, axis=-1, keepdims=True)
    e0 = jnp.min(jnp.where(g == v1, iota, E), axis=-1, keepdims=True)
    oh0 = (iota == e0)
    g2 = jnp.where(oh0, -1e30, g)
    v2 = jnp.max(g2, axis=-1, keepdims=True)
    e1 = jnp.min(jnp.where(g2 == v2, iota, E), axis=-1, keepdims=True)
    oh1 = (iota == e1)
    w0 = 1.0 / (1.0 + jnp.exp(v2 - v1))
    w0_out[...] = w0
    w1_out[...] = 1.0 - w0
    e0_out[...] = e0
    e1_out[...] = e1

    r_iota = lax.broadcasted_iota(jnp.int32, (TB, TB), 0)
    c_iota = lax.broadcasted_iota(jnp.int32, (TB, TB), 1)
    Ls = (c_iota < r_iota).astype(jnp.float32)
    oh0f = oh0.astype(jnp.float32)
    oh1f = oh1.astype(jnp.float32)
    base = cnt_s[...]
    cs0 = jnp.dot(Ls, oh0f, preferred_element_type=jnp.float32)
    r0_out[...] = jnp.sum(oh0f * (cs0 + base), axis=-1, keepdims=True)
    base1 = base + jnp.sum(oh0f, axis=0, keepdims=True)
    cs1 = jnp.dot(Ls, oh1f, preferred_element_type=jnp.float32)
    r1_out[...] = jnp.sum(oh1f * (cs1 + base1), axis=-1, keepdims=True)
    newcnt = base1 + jnp.sum(oh1f, axis=0, keepdims=True)
    cnt_s[...] = newcnt
    cnt_out[...] = newcnt


def _post_call(attn, Wo, b_o, hidden_states, ln2_w, Wr, br):
    return pl.pallas_call(
        _post_body,
        grid=(NTB,),
        in_specs=[
            pl.BlockSpec((TB, Q_SIZE), lambda i: (i, 0)),
            pl.BlockSpec((Q_SIZE, HIDDEN), lambda i: (0, 0)),
            pl.BlockSpec((1, HIDDEN), lambda i: (0, 0)),
            pl.BlockSpec((TB, HIDDEN), lambda i: (i, 0)),
            pl.BlockSpec((1, HIDDEN), lambda i: (0, 0)),
            pl.BlockSpec((HIDDEN, E), lambda i: (0, 0)),
            pl.BlockSpec((1, E), lambda i: (0, 0)),
        ],
        out_specs=[
            pl.BlockSpec((TB, HIDDEN), lambda i: (i, 0)),
            pl.BlockSpec((TB, HIDDEN), lambda i: (i, 0)),
            pl.BlockSpec((TB, 1), lambda i: (i, 0)),
            pl.BlockSpec((TB, 1), lambda i: (i, 0)),
            pl.BlockSpec((TB, 1), lambda i: (i, 0)),
            pl.BlockSpec((TB, 1), lambda i: (i, 0)),
            pl.BlockSpec((TB, 1), lambda i: (i, 0)),
            pl.BlockSpec((TB, 1), lambda i: (i, 0)),
            pl.BlockSpec((1, E), lambda i: (0, 0)),
        ],
        out_shape=[
            jax.ShapeDtypeStruct((T, HIDDEN), jnp.float32),
            jax.ShapeDtypeStruct((T, HIDDEN), jnp.float32),
            jax.ShapeDtypeStruct((T, 1), jnp.int32),
            jax.ShapeDtypeStruct((T, 1), jnp.int32),
            jax.ShapeDtypeStruct((T, 1), jnp.float32),
            jax.ShapeDtypeStruct((T, 1), jnp.float32),
            jax.ShapeDtypeStruct((T, 1), jnp.float32),
            jax.ShapeDtypeStruct((T, 1), jnp.float32),
            jax.ShapeDtypeStruct((1, E), jnp.float32),
        ],
        scratch_shapes=[pltpu.VMEM((1, E), jnp.float32)],
    )(attn, Wo, b_o, hidden_states, ln2_w, Wr, br)


def _disp_body(cnt, e0, e1, r0, r1, p0_out, p1_out, be_out):
    i = pl.program_id(0)
    c = cnt[...]
    pc = jnp.ceil(c * (1.0 / MB)) * MB
    eu = lax.broadcasted_iota(jnp.int32, (E, E), 0)
    ev = lax.broadcasted_iota(jnp.int32, (E, E), 1)
    U = (eu < ev).astype(jnp.float32)
    po = jnp.dot(pc, U, preferred_element_type=jnp.float32)

    iota = lax.broadcasted_iota(jnp.int32, (TB, E), 1)
    oh0 = (iota == e0[...]).astype(jnp.float32)
    oh1 = (iota == e1[...]).astype(jnp.float32)
    p0_out[...] = (jnp.sum(oh0 * po, axis=-1, keepdims=True) + r0[...]).astype(jnp.int32)
    p1_out[...] = (jnp.sum(oh1 * po, axis=-1, keepdims=True) + r1[...]).astype(jnp.int32)

    @pl.when(i == 0)
    def _():
        jB = lax.broadcasted_iota(jnp.int32, (NMB, E), 0).astype(jnp.float32) * MB
        poB = jnp.broadcast_to(po, (NMB, E))
        pcB = jnp.broadcast_to(pc, (NMB, E))
        ind = ((poB <= jB) & (jB < poB + pcB)).astype(jnp.float32)
        e_iota = lax.broadcasted_iota(jnp.int32, (NMB, E), 1).astype(jnp.float32)
        total = jnp.sum(pc)
        be = jnp.sum(ind * e_iota, axis=-1, keepdims=True)
        be = jnp.where(jB[:, 0:1] >= total, float(E - 1), be)
        be_out[...] = be.astype(jnp.int32)


def _disp_call(cnt, e0, e1, r0, r1):
    return pl.pallas_call(
        _disp_body,
        grid=(NTB,),
        in_specs=[
            pl.BlockSpec((1, E), lambda i: (0, 0)),
            pl.BlockSpec((TB, 1), lambda i: (i, 0)),
            pl.BlockSpec((TB, 1), lambda i: (i, 0)),
            pl.BlockSpec((TB, 1), lambda i: (i, 0)),
            pl.BlockSpec((TB, 1), lambda i: (i, 0)),
        ],
        out_specs=[
            pl.BlockSpec((TB, 1), lambda i: (i, 0)),
            pl.BlockSpec((TB, 1), lambda i: (i, 0)),
            pl.BlockSpec((NMB, 1), lambda i: (0, 0)),
        ],
        out_shape=[
            jax.ShapeDtypeStruct((T, 1), jnp.int32),
            jax.ShapeDtypeStruct((T, 1), jnp.int32),
            jax.ShapeDtypeStruct((NMB, 1), jnp.int32),
        ],
    )(cnt, e0, e1, r0, r1)


def _moe_body(be_ref, xs, w1r, b1r, w2r, b2r, y_out):
    x = xs[...]
    w1 = w1r[0]
    hh = jnp.dot(x, w1, preferred_element_type=jnp.float32) + b1r[0]
    gate = jnp.minimum(hh[:, :INTER], 7.0)
    up = jnp.clip(hh[:, INTER:], -7.0, 7.0)
    glu = gate / (1.0 + jnp.exp(-1.702 * gate))
    act = (up + 1.0) * glu
    y_out[...] = jnp.dot(act, w2r[0], preferred_element_type=jnp.float32) + b2r[0]


def _moe_call(be, x_sorted, w1, b1, w2, b2):
    grid_spec = pltpu.PrefetchScalarGridSpec(
        num_scalar_prefetch=1,
        grid=(NMB,),
        in_specs=[
            pl.BlockSpec((MB, HIDDEN), lambda j, be: (j, 0)),
            pl.BlockSpec((1, HIDDEN, 2 * INTER), lambda j, be: (be[j], 0, 0)),
            pl.BlockSpec((1, 1, 2 * INTER), lambda j, be: (be[j], 0, 0)),
            pl.BlockSpec((1, HIDDEN, HIDDEN), lambda j, be: (be[j], 0, 0)),
            pl.BlockSpec((1, 1, HIDDEN), lambda j, be: (be[j], 0, 0)),
        ],
        out_specs=pl.BlockSpec((MB, HIDDEN), lambda j, be: (j, 0)),
    )
    return pl.pallas_call(
        _moe_body,
        grid_spec=grid_spec,
        out_shape=jax.ShapeDtypeStruct((N_PAD, HIDDEN), jnp.float32),
        compiler_params=pltpu.CompilerParams(
            dimension_semantics=("arbitrary",),
        ),
    )(be, x_sorted, w1, b1[:, None, :], w2, b2[:, None, :])


@jax.jit
def _run(position_ids, hidden_states, ln1_w, Wqkv, b_qkv, sinks, Wo, b_o,
         ln2_w, Wr, br, w1, b1, w2, b2):
    inv_freq = _yarn_inv_freq()
    mscale = 0.1 * math.log(FACTOR) + 1.0
    t = position_ids.astype(jnp.float32)[:, None] * inv_freq[None, :]
    cos = jnp.cos(t) * mscale
    sin = jnp.sin(t) * mscale

    ln1_2d = ln1_w[None, :]
    ln2_2d = ln2_w[None, :]
    bqkv_2d = b_qkv[None, :]
    bo_2d = b_o[None, :]
    br_2d = br[None, :]
    sinks_2d = sinks[None, :]

    q, k, v = _qkv_call(hidden_states, ln1_2d, Wqkv, bqkv_2d, cos, sin)
    attn = _attn_call(q, k, v, sinks_2d)
    (res2, h2, e0, e1, r0, r1, w0, w1w, cnt) = _post_call(
        attn, Wo, bo_2d, hidden_states, ln2_2d, Wr, br_2d)
    p0, p1, be = _disp_call(cnt, e0, e1, r0, r1)

    p0f = p0[:, 0]
    p1f = p1[:, 0]
    x_sorted = jnp.zeros((N_PAD, HIDDEN), jnp.float32)
    x_sorted = x_sorted.at[p0f].set(h2).at[p1f].set(h2)

    y_sorted = _moe_call(be[:, 0], x_sorted, w1, b1, w2, b2)

    out = w0 * y_sorted[p0f] + w1w * y_sorted[p1f]
    return out, res2


def kernel(position_ids, hidden_states, ln1_w, Wqkv, b_qkv, sinks, Wo, b_o,
           ln2_w, Wr, br, w1, b1, w2, b2):
    return _run(position_ids, hidden_states, ln1_w, Wqkv, b_qkv, sinks,
                Wo, b_o, ln2_w, Wr, br, w1, b1, w2, b2)


# speedup vs baseline: 2.1274x; 1.1058x over previous
import functools
import math

import jax
import jax.numpy as jnp
from jax import lax
from jax.experimental import pallas as pl
from jax.experimental.pallas import tpu as pltpu
from jax.experimental.pallas import tpu_sc as plsc

T = 2048
HIDDEN = 1024
N_HEADS = 16
N_KV = 4
HEAD_DIM = 64
Q_SIZE = N_HEADS * HEAD_DIM
KV_SIZE = N_KV * HEAD_DIM
E = 16
TOP_K = 2
INTER = 1024
EPS = 1e-06
ROPE_THETA = 150000.0
FACTOR = 32.0
ORIG_MAX = 4096
BETA_FAST = 32.0
BETA_SLOW = 1.0
WINDOW = 128

TB = 256
NTB = T // TB
MB = 128
N_PAD = TOP_K * T + E * MB
NMB = N_PAD // MB
HALF = HEAD_DIM // 2


def _yarn_inv_freq():
    half = HEAD_DIM // 2
    pos_freqs = ROPE_THETA ** (jnp.arange(0, half, dtype=jnp.float32) * 2.0 / HEAD_DIM)
    inv_freq = 1.0 / pos_freqs
    inv_freq_inter = inv_freq / FACTOR

    def corr_dim(r):
        return HEAD_DIM * math.log(ORIG_MAX / (r * 2.0 * math.pi)) / (2.0 * math.log(ROPE_THETA))

    low = max(math.floor(corr_dim(BETA_FAST)), 0)
    high = min(math.ceil(corr_dim(BETA_SLOW)), half - 1)
    ramp = jnp.clip((jnp.arange(half, dtype=jnp.float32) - low) / max(high - low, 1), 0.0, 1.0)
    mask = 1.0 - ramp
    return inv_freq * mask + inv_freq_inter * (1.0 - mask)


def _qkv_body(hs, ln1, wqkv, bqkv, cos, sin, q_out, k_out, v_out):
    x = hs[...]
    var = jnp.mean(x * x, axis=-1, keepdims=True)
    h = x * lax.rsqrt(var + EPS) * ln1[...]
    qkv = jnp.dot(h, wqkv[...], preferred_element_type=jnp.float32) + bqkv[...]
    c = cos[...]
    s = sin[...]
    for hd in range(N_HEADS):
        base = hd * HEAD_DIM
        x1 = qkv[:, base:base + HALF]
        x2 = qkv[:, base + HALF:base + HEAD_DIM]
        q_out[:, base:base + HALF] = x1 * c - x2 * s
        q_out[:, base + HALF:base + HEAD_DIM] = x2 * c + x1 * s
    for hd in range(N_KV):
        base = Q_SIZE + hd * HEAD_DIM
        ob = hd * HEAD_DIM
        x1 = qkv[:, base:base + HALF]
        x2 = qkv[:, base + HALF:base + HEAD_DIM]
        k_out[:, ob:ob + HALF] = x1 * c - x2 * s
        k_out[:, ob + HALF:ob + HEAD_DIM] = x2 * c + x1 * s
    v_out[...] = qkv[:, Q_SIZE + KV_SIZE:]


def _qkv_call(hidden_states, ln1_w, Wqkv, b_qkv, cos, sin):
    return pl.pallas_call(
        _qkv_body,
        grid=(NTB,),
        in_specs=[
            pl.BlockSpec((TB, HIDDEN), lambda i: (i, 0)),
            pl.BlockSpec((1, HIDDEN), lambda i: (0, 0)),
            pl.BlockSpec((HIDDEN, Q_SIZE + 2 * KV_SIZE), lambda i: (0, 0)),
            pl.BlockSpec((1, Q_SIZE + 2 * KV_SIZE), lambda i: (0, 0)),
            pl.BlockSpec((TB, HALF), lambda i: (i, 0)),
            pl.BlockSpec((TB, HALF), lambda i: (i, 0)),
        ],
        out_specs=[
            pl.BlockSpec((TB, Q_SIZE), lambda i: (i, 0)),
            pl.BlockSpec((TB, KV_SIZE), lambda i: (i, 0)),
            pl.BlockSpec((TB, KV_SIZE), lambda i: (i, 0)),
        ],
        out_shape=[
            jax.ShapeDtypeStruct((T, Q_SIZE), jnp.float32),
            jax.ShapeDtypeStruct((T, KV_SIZE), jnp.float32),
            jax.ShapeDtypeStruct((T, KV_SIZE), jnp.float32),
        ],
    )(hidden_states, ln1_w, Wqkv, b_qkv, cos, sin)


def _attn_body(q, kp, kc, kn, vp, vc, vn, sinks, out):
    qb = pl.program_id(0)
    scale = HEAD_DIM ** -0.5
    i_abs = qb * TB + lax.broadcasted_iota(jnp.int32, (TB, 3 * 128), 0)
    j_abs = (2 * qb - 1) * 128 + lax.broadcasted_iota(jnp.int32, (TB, 3 * 128), 1)
    mask = (j_abs >= 0) & (j_abs <= i_abs) & (i_abs - j_abs < WINDOW)
    for hd in range(N_HEADS):
        hk = hd // (N_HEADS // N_KV)
        qh = q[:, hd * HEAD_DIM:(hd + 1) * HEAD_DIM]
        kh = jnp.concatenate(
            [kp[:, hk * HEAD_DIM:(hk + 1) * HEAD_DIM],
             kc[:, hk * HEAD_DIM:(hk + 1) * HEAD_DIM],
             kn[:, hk * HEAD_DIM:(hk + 1) * HEAD_DIM]], axis=0)
        vh = jnp.concatenate(
            [vp[:, hk * HEAD_DIM:(hk + 1) * HEAD_DIM],
             vc[:, hk * HEAD_DIM:(hk + 1) * HEAD_DIM],
             vn[:, hk * HEAD_DIM:(hk + 1) * HEAD_DIM]], axis=0)
        scores = lax.dot_general(qh, kh, (((1,), (1,)), ((), ())),
                                 preferred_element_type=jnp.float32) * scale
        scores = jnp.where(mask, scores, -1e30)
        sink = sinks[0:1, hd:hd + 1]
        m = jnp.maximum(jnp.max(scores, axis=-1, keepdims=True), sink)
        e = jnp.exp(scores - m)
        denom = jnp.sum(e, axis=-1, keepdims=True) + jnp.exp(sink - m)
        probs = e / denom
        out[:, hd * HEAD_DIM:(hd + 1) * HEAD_DIM] = jnp.dot(
            probs, vh, preferred_element_type=jnp.float32)


def _attn_call(q, k, v, sinks):
    kv_spec_p = pl.BlockSpec((128, KV_SIZE), lambda i: (jnp.maximum(2 * i - 1, 0), 0))
    kv_spec_c = pl.BlockSpec((128, KV_SIZE), lambda i: (2 * i, 0))
    kv_spec_n = pl.BlockSpec((128, KV_SIZE), lambda i: (2 * i + 1, 0))
    return pl.pallas_call(
        _attn_body,
        grid=(NTB,),
        in_specs=[
            pl.BlockSpec((TB, Q_SIZE), lambda i: (i, 0)),
            kv_spec_p, kv_spec_c, kv_spec_n,
            kv_spec_p, kv_spec_c, kv_spec_n,
            pl.BlockSpec((1, N_HEADS), lambda i: (0, 0)),
        ],
        out_specs=pl.BlockSpec((TB, Q_SIZE), lambda i: (i, 0)),
        out_shape=jax.ShapeDtypeStruct((T, Q_SIZE), jnp.float32),
    )(q, k, k, k, v, v, v, sinks)


def _post_body(attn, wo, bo, hs, ln2, wr, br,
               res2_out, h2_out, e0_out, e1_out, r0_out, r1_out,
               w0_out, w1_out, cnt_out, cnt_s):
    i = pl.program_id(0)

    @pl.when(i == 0)
    def _():
        cnt_s[...] = jnp.zeros_like(cnt_s)

    o = jnp.dot(attn[...], wo[...], preferred_element_type=jnp.float32) + bo[...]
    res2 = o + hs[...]
    res2_out[...] = res2
    var = jnp.mean(res2 * res2, axis=-1, keepdims=True)
    h2 = res2 * lax.rsqrt(var + EPS) * ln2[...]
    h2_out[...] = h2
    g = jnp.dot(h2, wr[...], preferred_element_type=jnp.float32) + br[...]

    iota = lax.broadcasted_iota(jnp.int32, (TB, E), 1)
    v1 = jnp.max(g, axis=-1, keepdims=True)
    e0 = jnp.min(jnp.where(g == v1, iota, E), axis=-1, keepdims=True)
    oh0 = (iota == e0)
    g2 = jnp.where(oh0, -1e30, g)
    v2 = jnp.max(g2, axis=-1, keepdims=True)
    e1 = jnp.min(jnp.where(g2 == v2, iota, E), axis=-1, keepdims=True)
    oh1 = (iota == e1)
    w0 = 1.0 / (1.0 + jnp.exp(v2 - v1))
    w0_out[...] = w0
    w1_out[...] = 1.0 - w0
    e0_out[...] = e0
    e1_out[...] = e1

    r_iota = lax.broadcasted_iota(jnp.int32, (TB, TB), 0)
    c_iota = lax.broadcasted_iota(jnp.int32, (TB, TB), 1)
    Ls = (c_iota < r_iota).astype(jnp.float32)
    oh0f = oh0.astype(jnp.float32)
    oh1f = oh1.astype(jnp.float32)
    base = cnt_s[...]
    cs0 = jnp.dot(Ls, oh0f, preferred_element_type=jnp.float32)
    r0_out[...] = jnp.sum(oh0f * (cs0 + base), axis=-1, keepdims=True)
    base1 = base + jnp.sum(oh0f, axis=0, keepdims=True)
    cs1 = jnp.dot(Ls, oh1f, preferred_element_type=jnp.float32)
    r1_out[...] = jnp.sum(oh1f * (cs1 + base1), axis=-1, keepdims=True)
    newcnt = base1 + jnp.sum(oh1f, axis=0, keepdims=True)
    cnt_s[...] = newcnt
    cnt_out[...] = newcnt


def _post_call(attn, Wo, b_o, hidden_states, ln2_w, Wr, br):
    return pl.pallas_call(
        _post_body,
        grid=(NTB,),
        in_specs=[
            pl.BlockSpec((TB, Q_SIZE), lambda i: (i, 0)),
            pl.BlockSpec((Q_SIZE, HIDDEN), lambda i: (0, 0)),
            pl.BlockSpec((1, HIDDEN), lambda i: (0, 0)),
            pl.BlockSpec((TB, HIDDEN), lambda i: (i, 0)),
            pl.BlockSpec((1, HIDDEN), lambda i: (0, 0)),
            pl.BlockSpec((HIDDEN, E), lambda i: (0, 0)),
            pl.BlockSpec((1, E), lambda i: (0, 0)),
        ],
        out_specs=[
            pl.BlockSpec((TB, HIDDEN), lambda i: (i, 0)),
            pl.BlockSpec((TB, HIDDEN), lambda i: (i, 0)),
            pl.BlockSpec((TB, 1), lambda i: (i, 0)),
            pl.BlockSpec((TB, 1), lambda i: (i, 0)),
            pl.BlockSpec((TB, 1), lambda i: (i, 0)),
            pl.BlockSpec((TB, 1), lambda i: (i, 0)),
            pl.BlockSpec((TB, 1), lambda i: (i, 0)),
            pl.BlockSpec((TB, 1), lambda i: (i, 0)),
            pl.BlockSpec((1, E), lambda i: (0, 0)),
        ],
        out_shape=[
            jax.ShapeDtypeStruct((T, HIDDEN), jnp.float32),
            jax.ShapeDtypeStruct((T, HIDDEN), jnp.float32),
            jax.ShapeDtypeStruct((T, 1), jnp.int32),
            jax.ShapeDtypeStruct((T, 1), jnp.int32),
            jax.ShapeDtypeStruct((T, 1), jnp.float32),
            jax.ShapeDtypeStruct((T, 1), jnp.float32),
            jax.ShapeDtypeStruct((T, 1), jnp.float32),
            jax.ShapeDtypeStruct((T, 1), jnp.float32),
            jax.ShapeDtypeStruct((1, E), jnp.float32),
        ],
        scratch_shapes=[pltpu.VMEM((1, E), jnp.float32)],
    )(attn, Wo, b_o, hidden_states, ln2_w, Wr, br)


def _disp_body(cnt, e0, e1, r0, r1, p0_out, p1_out, be_out):
    i = pl.program_id(0)
    c = cnt[...]
    pc = jnp.ceil(c * (1.0 / MB)) * MB
    eu = lax.broadcasted_iota(jnp.int32, (E, E), 0)
    ev = lax.broadcasted_iota(jnp.int32, (E, E), 1)
    U = (eu < ev).astype(jnp.float32)
    po = jnp.dot(pc, U, preferred_element_type=jnp.float32)

    iota = lax.broadcasted_iota(jnp.int32, (TB, E), 1)
    oh0 = (iota == e0[...]).astype(jnp.float32)
    oh1 = (iota == e1[...]).astype(jnp.float32)
    p0_out[...] = (jnp.sum(oh0 * po, axis=-1, keepdims=True) + r0[...]).astype(jnp.int32)
    p1_out[...] = (jnp.sum(oh1 * po, axis=-1, keepdims=True) + r1[...]).astype(jnp.int32)

    @pl.when(i == 0)
    def _():
        jB = lax.broadcasted_iota(jnp.int32, (NMB, E), 0).astype(jnp.float32) * MB
        poB = jnp.broadcast_to(po, (NMB, E))
        pcB = jnp.broadcast_to(pc, (NMB, E))
        ind = ((poB <= jB) & (jB < poB + pcB)).astype(jnp.float32)
        e_iota = lax.broadcasted_iota(jnp.int32, (NMB, E), 1).astype(jnp.float32)
        total = jnp.sum(pc)
        be = jnp.sum(ind * e_iota, axis=-1, keepdims=True)
        be = jnp.where(jB[:, 0:1] >= total, float(E - 1), be)
        be_out[...] = be.astype(jnp.int32)


def _disp_call(cnt, e0, e1, r0, r1):
    return pl.pallas_call(
        _disp_body,
        grid=(NTB,),
        in_specs=[
            pl.BlockSpec((1, E), lambda i: (0, 0)),
            pl.BlockSpec((TB, 1), lambda i: (i, 0)),
            pl.BlockSpec((TB, 1), lambda i: (i, 0)),
            pl.BlockSpec((TB, 1), lambda i: (i, 0)),
            pl.BlockSpec((TB, 1), lambda i: (i, 0)),
        ],
        out_specs=[
            pl.BlockSpec((TB, 1), lambda i: (i, 0)),
            pl.BlockSpec((TB, 1), lambda i: (i, 0)),
            pl.BlockSpec((NMB, 1), lambda i: (0, 0)),
        ],
        out_shape=[
            jax.ShapeDtypeStruct((T, 1), jnp.int32),
            jax.ShapeDtypeStruct((T, 1), jnp.int32),
            jax.ShapeDtypeStruct((NMB, 1), jnp.int32),
        ],
    )(cnt, e0, e1, r0, r1)


def _moe_body(be_ref, xs, w1r, b1r, w2r, b2r, y_out):
    x = xs[...].astype(jnp.bfloat16)
    w1 = w1r[0].astype(jnp.bfloat16)
    hh = jnp.dot(x, w1, preferred_element_type=jnp.float32) + b1r[0]
    gate = jnp.minimum(hh[:, :INTER], 7.0)
    up = jnp.clip(hh[:, INTER:], -7.0, 7.0)
    glu = gate / (1.0 + jnp.exp(-1.702 * gate))
    act = ((up + 1.0) * glu).astype(jnp.bfloat16)
    y_out[...] = jnp.dot(act, w2r[0].astype(jnp.bfloat16),
                         preferred_element_type=jnp.float32) + b2r[0]


def _moe_call(be, x_sorted, w1, b1, w2, b2):
    grid_spec = pltpu.PrefetchScalarGridSpec(
        num_scalar_prefetch=1,
        grid=(NMB,),
        in_specs=[
            pl.BlockSpec((MB, HIDDEN), lambda j, be: (j, 0)),
            pl.BlockSpec((1, HIDDEN, 2 * INTER), lambda j, be: (be[j], 0, 0)),
            pl.BlockSpec((1, 1, 2 * INTER), lambda j, be: (be[j], 0, 0)),
            pl.BlockSpec((1, HIDDEN, HIDDEN), lambda j, be: (be[j], 0, 0)),
            pl.BlockSpec((1, 1, HIDDEN), lambda j, be: (be[j], 0, 0)),
        ],
        out_specs=pl.BlockSpec((MB, HIDDEN), lambda j, be: (j, 0)),
    )
    return pl.pallas_call(
        _moe_body,
        grid_spec=grid_spec,
        out_shape=jax.ShapeDtypeStruct((N_PAD, HIDDEN), jnp.float32),
        compiler_params=pltpu.CompilerParams(
            dimension_semantics=("arbitrary",),
        ),
    )(be, x_sorted, w1, b1[:, None, :], w2, b2[:, None, :])


_SC_MESH = plsc.VectorSubcoreMesh(core_axis_name="c", subcore_axis_name="s")
_NW = 32
_RT = T // _NW
_CC = 32


@functools.partial(
    pl.kernel, mesh=_SC_MESH,
    out_type=jax.ShapeDtypeStruct((N_PAD, HIDDEN), jnp.float32),
    scratch_types=[
        pltpu.VMEM((_RT,), jnp.int32),
        pltpu.VMEM((_RT,), jnp.int32),
        pltpu.VMEM((_RT, HIDDEN), jnp.float32),
        pltpu.SemaphoreType.DMA,
    ],
)
def _sc_scatter(h2_hbm, p0_hbm, p1_hbm, out_hbm, i0_v, i1_v, rows_v, sem):
    wid = lax.axis_index("s") * 2 + lax.axis_index("c")
    base = wid * _RT
    pltpu.sync_copy(p0_hbm.at[pl.ds(base, _RT)], i0_v)
    pltpu.sync_copy(p1_hbm.at[pl.ds(base, _RT)], i1_v)
    pltpu.sync_copy(h2_hbm.at[pl.ds(base, _RT)], rows_v)
    pltpu.async_copy(rows_v, out_hbm.at[i0_v], sem).wait()
    pltpu.async_copy(rows_v, out_hbm.at[i1_v], sem).wait()


@functools.partial(
    pl.kernel, mesh=_SC_MESH,
    out_type=(
        jax.ShapeDtypeStruct((T, HIDDEN), jnp.float32),
        jax.ShapeDtypeStruct((T, HIDDEN), jnp.float32),
    ),
    scratch_types=[
        pltpu.VMEM((_CC,), jnp.int32),
        pltpu.VMEM((_CC,), jnp.int32),
        pltpu.VMEM((_CC, HIDDEN), jnp.float32),
        pltpu.VMEM((_CC, HIDDEN), jnp.float32),
        pltpu.SemaphoreType.DMA,
    ],
)
def _sc_gather2(y_hbm, p0_hbm, p1_hbm, out0_hbm, out1_hbm,
                i0_v, i1_v, y0_v, y1_v, sem):
    wid = lax.axis_index("s") * 2 + lax.axis_index("c")
    for cnk in range(_RT // _CC):
        base = wid * _RT + cnk * _CC
        pltpu.sync_copy(p0_hbm.at[pl.ds(base, _CC)], i0_v)
        pltpu.sync_copy(p1_hbm.at[pl.ds(base, _CC)], i1_v)
        cp0 = pltpu.async_copy(y_hbm.at[i0_v], y0_v, sem)
        cp1 = pltpu.async_copy(y_hbm.at[i1_v], y1_v, sem)
        cp0.wait()
        cp1.wait()
        pltpu.sync_copy(y0_v, out0_hbm.at[pl.ds(base, _CC)])
        pltpu.sync_copy(y1_v, out1_hbm.at[pl.ds(base, _CC)])


def _comb_body(y0, y1, w0, w1, out):
    out[...] = w0[...] * y0[...] + w1[...] * y1[...]


def _comb_call(y0, y1, w0, w1):
    return pl.pallas_call(
        _comb_body,
        grid=(NTB,),
        in_specs=[
            pl.BlockSpec((TB, HIDDEN), lambda i: (i, 0)),
            pl.BlockSpec((TB, HIDDEN), lambda i: (i, 0)),
            pl.BlockSpec((TB, 1), lambda i: (i, 0)),
            pl.BlockSpec((TB, 1), lambda i: (i, 0)),
        ],
        out_specs=pl.BlockSpec((TB, HIDDEN), lambda i: (i, 0)),
        out_shape=jax.ShapeDtypeStruct((T, HIDDEN), jnp.float32),
    )(y0, y1, w0, w1)


@jax.jit
def _run(position_ids, hidden_states, ln1_w, Wqkv, b_qkv, sinks, Wo, b_o,
         ln2_w, Wr, br, w1, b1, w2, b2):
    inv_freq = _yarn_inv_freq()
    mscale = 0.1 * math.log(FACTOR) + 1.0
    t = position_ids.astype(jnp.float32)[:, None] * inv_freq[None, :]
    cos = jnp.cos(t) * mscale
    sin = jnp.sin(t) * mscale

    ln1_2d = ln1_w[None, :]
    ln2_2d = ln2_w[None, :]
    bqkv_2d = b_qkv[None, :]
    bo_2d = b_o[None, :]
    br_2d = br[None, :]
    sinks_2d = sinks[None, :]

    q, k, v = _qkv_call(hidden_states, ln1_2d, Wqkv, bqkv_2d, cos, sin)
    attn = _attn_call(q, k, v, sinks_2d)
    (res2, h2, e0, e1, r0, r1, w0, w1w, cnt) = _post_call(
        attn, Wo, bo_2d, hidden_states, ln2_2d, Wr, br_2d)
    p0, p1, be = _disp_call(cnt, e0, e1, r0, r1)

    p0f = p0[:, 0]
    p1f = p1[:, 0]
    x_sorted = _sc_scatter(h2, p0f, p1f)
    y_sorted = _moe_call(be[:, 0], x_sorted, w1, b1, w2, b2)
    y0g, y1g = _sc_gather2(y_sorted, p0f, p1f)
    out = _comb_call(y0g, y1g, w0, w1w)
    return out, res2


def kernel(position_ids, hidden_states, ln1_w, Wqkv, b_qkv, sinks, Wo, b_o,
           ln2_w, Wr, br, w1, b1, w2, b2):
    return _run(position_ids, hidden_states, ln1_w, Wqkv, b_qkv, sinks,
                Wo, b_o, ln2_w, Wr, br, w1, b1, w2, b2)
